# baseline (speedup 1.0000x reference)
import functools

import jax
import jax.numpy as jnp
from jax import lax
from jax.experimental import pallas as pl
from jax.experimental.pallas import tpu as pltpu
from jax.experimental.pallas import tpu_sc as plsc

B, L = 4096, 200
V, D = 100000, 2
N = B * L
NC, NS = 2, 16
NSH = 16
PER_SH = N // NSH
NP = 16
P = PER_SH // NP
HP = P // 2
GRP = P // 16
SPM = 2 * 8 * 2 * P


def _make_kernel():
    mesh = plsc.VectorSubcoreMesh(core_axis_name="c", subcore_axis_name="s")

    @functools.partial(
        pl.kernel,
        out_type=jax.ShapeDtypeStruct((N * D,), jnp.float32),
        mesh=mesh,
        scratch_types=[
            pltpu.VMEM((V,), jnp.float32),
            pltpu.VMEM((P,), jnp.int32),
            pltpu.VMEM((P,), jnp.int32),
            pltpu.VMEM((P,), jnp.float32),
            pltpu.VMEM((P,), jnp.float32),
            pltpu.VMEM((HP,), jnp.float32),
            pltpu.VMEM((P,), jnp.float32),
            pltpu.VMEM((P,), jnp.float32),
            pltpu.VMEM_SHARED((SPM,), jnp.float32),
            pltpu.SemaphoreType.DMA,
            pltpu.SemaphoreType.DMA,
            pltpu.SemaphoreType.DMA,
        ],
        compiler_params=pltpu.CompilerParams(needs_layout_passes=False),
    )
    def k(x_hbm, wt_hbm, out_hbm, col_v, idx0, idx1, val0, val1,
          part_v, ost0, ost1, spm, sem_i, sem_x, sem_o):
        c = lax.axis_index("c")
        s = lax.axis_index("s")
        col = s % 2
        tloc = s // 2
        shard = c * 8 + tloc
        idx_bufs = (idx0, idx1)
        val_bufs = (val0, val1)
        ost_bufs = (ost0, ost1)
        iota2 = lax.iota(jnp.int32, 16) * 2

        first_idx = pltpu.async_copy(x_hbm.at[shard, 0], idx0, sem_i)
        pltpu.sync_copy(wt_hbm.at[col], col_v)

        idx_cps = [first_idx]
        out_cps = []
        for p in range(NP):
            cur = p % 2
            if p + 1 < NP:
                idx_cps.append(
                    pltpu.async_copy(x_hbm.at[shard, p + 1], idx_bufs[1 - cur], sem_i)
                )
            idx_cps[p].wait()
            if p >= 2:
                out_cps[p - 2].wait()

            ib = idx_bufs[cur]
            vb = val_bufs[cur]
            ob = ost_bufs[cur]

            def gather_block(i, _):
                for u in range(8):
                    o = i * 128 + u * 16
                    vb[pl.ds(o, 16)] = plsc.load_gather(col_v, [ib[pl.ds(o, 16)]])
                return 0

            lax.fori_loop(0, GRP // 8, gather_block, 0)

            my_off = ((col * 8 + tloc) * 2 + cur) * P
            pltpu.async_copy(vb, spm.at[pl.ds(my_off, P)], sem_x).wait()
            plsc.subcore_barrier()

            half = col
            pa_off = (((1 - col) * 8 + tloc) * 2 + cur) * P + half * HP
            pltpu.sync_copy(spm.at[pl.ds(pa_off, HP)], part_v)

            def ileave_block(g, _):
                own = vb[pl.ds(half * HP + g * 16, 16)]
                par = part_v[pl.ds(g * 16, 16)]
                pos = iota2 + g * 32
                plsc.store_scatter(ob, [pos + col], own)
                plsc.store_scatter(ob, [pos + (1 - col)], par)
                return 0

            lax.fori_loop(0, HP // 16, ileave_block, 0)

            flat = (shard * PER_SH + p * P + half * HP) * D
            out_cps.append(
                pltpu.async_copy(ob, out_hbm.at[pl.ds(flat, P)], sem_o)
            )
        out_cps[NP - 2].wait()
        out_cps[NP - 1].wait()

    return k


_gather = _make_kernel()


def kernel(x, W):
    xr = x.reshape(NSH, NP, P)
    wt = W.T.reshape(NC, V)
    out = _gather(xr, wt)
    return out.reshape(B, L, D)


# speedup vs baseline: 7.6583x; 7.6583x over previous
import functools

import jax
import jax.numpy as jnp
from jax import lax
from jax.experimental import pallas as pl
from jax.experimental.pallas import tpu as pltpu
from jax.experimental.pallas import tpu_sc as plsc

B, L = 4096, 200
V, D = 100000, 2
N = B * L
NC, NS = 2, 16
PER_S = N // NS
NP = 10
P = PER_S // NP
GRP = P // 16


def _make_kernel():
    mesh = plsc.VectorSubcoreMesh(core_axis_name="c", subcore_axis_name="s")

    @functools.partial(
        pl.kernel,
        out_type=jax.ShapeDtypeStruct((NC, N), jnp.float32),
        mesh=mesh,
        scratch_types=[
            pltpu.VMEM((V,), jnp.float32),
            pltpu.VMEM((P,), jnp.int32),
            pltpu.VMEM((P,), jnp.int32),
            pltpu.VMEM((P,), jnp.float32),
            pltpu.VMEM((P,), jnp.float32),
            pltpu.SemaphoreType.DMA,
            pltpu.SemaphoreType.DMA,
        ],
        compiler_params=pltpu.CompilerParams(needs_layout_passes=False),
    )
    def k(x_hbm, wt_hbm, out_hbm, col_v, idx0, idx1, val0, val1, sem_i, sem_o):
        c = lax.axis_index("c")
        s = lax.axis_index("s")
        idx_bufs = (idx0, idx1)
        val_bufs = (val0, val1)
        first_idx = pltpu.async_copy(x_hbm.at[s, 0], idx0, sem_i)
        pltpu.sync_copy(wt_hbm.at[c], col_v)

        idx_cps = [first_idx]
        out_cps = []
        for p in range(NP):
            cur = p % 2
            if p + 1 < NP:
                idx_cps.append(
                    pltpu.async_copy(x_hbm.at[s, p + 1], idx_bufs[1 - cur], sem_i)
                )
            idx_cps[p].wait()
            if p >= 2:
                out_cps[p - 2].wait()

            ib = idx_bufs[cur]
            vb = val_bufs[cur]

            def gather_block(i, _):
                for u in range(8):
                    o = i * 128 + u * 16
                    vb[pl.ds(o, 16)] = plsc.load_gather(col_v, [ib[pl.ds(o, 16)]])
                return 0

            lax.fori_loop(0, GRP // 8, gather_block, 0)
            base = s * PER_S + p * P
            out_cps.append(
                pltpu.async_copy(vb, out_hbm.at[c, pl.ds(base, P)], sem_o)
            )
        out_cps[NP - 2].wait()
        out_cps[NP - 1].wait()

    return k


_gather = _make_kernel()


def kernel(x, W):
    xr = x.reshape(NS, NP, P)
    wt = W.T.reshape(NC, V)
    out = _gather(xr, wt)
    return out.T.reshape(B, L, D)
